# Initial kernel scaffold; baseline (speedup 1.0000x reference)
import jax
import jax.numpy as jnp
from jax.experimental import pallas as pl


def kernel(query, attn_logits, pos_emb):
    raise NotImplementedError("write your pallas kernel here")


# speedup vs baseline: 7411.3882x; 7411.3882x over previous
import functools

import jax
import jax.numpy as jnp
from jax.experimental import pallas as pl
from jax.experimental.pallas import tpu as pltpu

_BQ = 256


def _cope_body(q_ref, a_ref, pe_ref, o_ref, *, skv: int, npos: int):
    t = jnp.dot(q_ref[...], pe_ref[...], preferred_element_type=jnp.float32)
    d = jnp.concatenate(
        [t[:, 1:] - t[:, :-1], jnp.zeros((t.shape[0], 1), jnp.float32)], axis=1
    )
    table = jnp.concatenate([t, d], axis=1)

    g = jax.nn.sigmoid(a_ref[...])
    lane = jax.lax.broadcasted_iota(jnp.int32, g.shape, 1)
    s = g
    sh = 1
    while sh < skv:
        shifted = pltpu.roll(s, skv - sh, axis=1)
        s = s + jnp.where(lane < skv - sh, shifted, 0.0)
        sh *= 2
    pos = jnp.minimum(s, float(npos - 1))
    pf = jnp.floor(pos)
    idx = pf.astype(jnp.int32)
    frac = pos - pf
    tv = jnp.take_along_axis(table, idx, axis=1)
    dv = jnp.take_along_axis(table, idx + npos, axis=1)
    o_ref[...] = tv + frac * dv


def kernel(query, attn_logits, pos_emb):
    b, h, sq, dim = query.shape
    skv = attn_logits.shape[-1]
    npos = pos_emb.shape[-1]
    rows = b * h * sq
    q2 = query.reshape(rows, dim)
    a2 = attn_logits.reshape(rows, skv)
    pe = pos_emb.reshape(dim, npos)

    body = functools.partial(_cope_body, skv=skv, npos=npos)
    out = pl.pallas_call(
        body,
        grid=(rows // _BQ,),
        in_specs=[
            pl.BlockSpec((_BQ, dim), lambda i: (i, 0)),
            pl.BlockSpec((_BQ, skv), lambda i: (i, 0)),
            pl.BlockSpec((dim, npos), lambda i: (0, 0)),
        ],
        out_specs=pl.BlockSpec((_BQ, skv), lambda i: (i, 0)),
        out_shape=jax.ShapeDtypeStruct((rows, skv), jnp.float32),
    )(q2, a2, pe)
    return out.reshape(b, h, sq, skv)
